# baseline (speedup 1.0000x reference)
import functools

import jax
import jax.numpy as jnp
from jax import lax
from jax.experimental import pallas as pl
from jax.experimental.pallas import tpu as pltpu
from jax.experimental.pallas import tpu_sc as plsc

_VOCAB = 1000
_VPAD = 1024
_ROWS_PER_BLOCK = 2048
_NSUB = 16


_LANES_PER_BLOCK = 2048


def _argmax_body(x0_ref, x1_ref, out_ref):
    _, vocab, bl = x0_ref.shape
    ids = lax.broadcasted_iota(jnp.int32, (vocab, bl), 0)
    for a, ref in enumerate((x0_ref, x1_ref)):
        xa = ref[0]
        m = jnp.max(xa, axis=0, keepdims=True)
        idx = jnp.min(jnp.where(xa == m, ids, vocab), axis=0)
        out_ref[pl.ds(a * bl, bl)] = idx


def _tc_argmax(utt_t):
    agents, vocab, batch = utt_t.shape
    nblk = batch // _LANES_PER_BLOCK
    blk = (1, vocab, _LANES_PER_BLOCK)
    out = pl.pallas_call(
        _argmax_body,
        grid=(nblk,),
        in_specs=[pl.BlockSpec(blk, lambda i: (0, 0, i)),
                  pl.BlockSpec(blk, lambda i: (1, 0, i))],
        out_specs=pl.BlockSpec((agents * _LANES_PER_BLOCK,), lambda i: (i,)),
        out_shape=jax.ShapeDtypeStruct((agents * batch,), jnp.int32),
    )(utt_t, utt_t)
    return out


def _sc_bincount(indices, wc1d, idx_per_sub):
    mesh = plsc.VectorSubcoreMesh(core_axis_name="c", subcore_axis_name="s")
    stripe = _VPAD // _NSUB

    @functools.partial(
        pl.kernel,
        mesh=mesh,
        out_type=jax.ShapeDtypeStruct((_VPAD,), jnp.float32),
        compiler_params=pltpu.CompilerParams(
            needs_layout_passes=False, use_tc_tiling_on_sc=False),
        scratch_types=[
            pltpu.VMEM((idx_per_sub,), jnp.int32),
            pltpu.VMEM((16 * _VPAD,), jnp.float32),
            pltpu.VMEM((_VPAD,), jnp.float32),
            pltpu.VMEM((_NSUB, _VPAD), jnp.float32),
            pltpu.VMEM((stripe,), jnp.float32),
            pltpu.VMEM_SHARED((_NSUB, _VPAD), jnp.float32),
            pltpu.SemaphoreType.DMA,
        ],
    )
    def bincount(idx_hbm, wc_hbm, out_hbm, idx_v, hist2, hist1, parts, ostripe,
                 shared, sem):
        core = lax.axis_index("c")
        sub = lax.axis_index("s")

        @pl.when(core == 0)
        def _():
            cp = pltpu.make_async_copy(
                idx_hbm.at[pl.ds(sub * idx_per_sub, idx_per_sub)], idx_v, sem)
            cp.start()
            zeros16 = jnp.zeros((16,), jnp.float32)

            def zero_body(g, c):
                hist2[pl.ds(g * 16, 16)] = zeros16
                return c
            lax.fori_loop(0, _VPAD, zero_body, 0)
            cp.wait()

            lane = lax.iota(jnp.int32, 16) * _VPAD
            ones16 = jnp.full((16,), 1.0, jnp.float32)

            def scat_body(g, c):
                iv = idx_v[pl.ds(g * 16, 16)]
                plsc.addupdate_scatter(hist2, [lane + iv], ones16)
                return c
            lax.fori_loop(0, idx_per_sub // 16, scat_body, 0)

            def red_body(c, carry):
                acc = hist2[pl.ds(c * 16, 16)]
                for l in range(1, 16):
                    acc = acc + hist2[pl.ds(l * _VPAD + c * 16, 16)]
                hist1[pl.ds(c * 16, 16)] = acc
                return carry
            lax.fori_loop(0, _VPAD // 16, red_body, 0)

            pltpu.sync_copy(hist1, shared.at[sub])
            plsc.subcore_barrier()
            pltpu.sync_copy(shared, parts)
            pltpu.sync_copy(wc_hbm.at[pl.ds(sub * stripe, stripe)], ostripe)

            def comb_body(r, carry):
                acc = ostripe[pl.ds(r * 16, 16)]
                for t in range(_NSUB):
                    acc = acc + parts[t, pl.ds(sub * stripe + r * 16, 16)]
                ostripe[pl.ds(r * 16, 16)] = acc
                return carry
            lax.fori_loop(0, stripe // 16, comb_body, 0)
            pltpu.sync_copy(ostripe, out_hbm.at[pl.ds(sub * stripe, stripe)])

    return bincount(indices, wc1d)


def kernel(utterances, word_counts):
    batch, agents, vocab = utterances.shape
    nrows = batch * agents
    utt_t = jnp.transpose(utterances, (1, 2, 0))
    indices = _tc_argmax(utt_t)
    wc1d = jnp.pad(word_counts, (0, _VPAD - vocab))
    hist = _sc_bincount(indices, wc1d, nrows // _NSUB)
    return hist[:vocab]


# speedup vs baseline: 1.0618x; 1.0618x over previous
import functools

import jax
import jax.numpy as jnp
from jax import lax
from jax.experimental import pallas as pl
from jax.experimental.pallas import tpu as pltpu
from jax.experimental.pallas import tpu_sc as plsc

_VOCAB = 1000
_VPAD = 1024
_ROWS_PER_BLOCK = 2048
_NSUB = 16


_LANES_PER_BLOCK = 2048


def _argmax_body(x0_ref, x1_ref, out_ref):
    _, vocab, bl = x0_ref.shape
    ids = lax.broadcasted_iota(jnp.int32, (vocab, bl), 0)
    for a, ref in enumerate((x0_ref, x1_ref)):
        xa = ref[0]
        m = jnp.max(xa, axis=0, keepdims=True)
        idx = jnp.min(jnp.where(xa == m, ids, vocab), axis=0)
        out_ref[pl.ds(a * bl, bl)] = idx


def _tc_argmax(utt_t):
    agents, vocab, batch = utt_t.shape
    nblk = batch // _LANES_PER_BLOCK
    blk = (1, vocab, _LANES_PER_BLOCK)
    out = pl.pallas_call(
        _argmax_body,
        grid=(nblk,),
        in_specs=[pl.BlockSpec(blk, lambda i: (0, 0, i)),
                  pl.BlockSpec(blk, lambda i: (1, 0, i))],
        out_specs=pl.BlockSpec((agents * _LANES_PER_BLOCK,), lambda i: (i,)),
        out_shape=jax.ShapeDtypeStruct((agents * batch,), jnp.int32),
    )(utt_t, utt_t)
    return out


def _sc_bincount(indices, wc1d, idx_per_sub):
    mesh = plsc.VectorSubcoreMesh(core_axis_name="c", subcore_axis_name="s")
    stripe = _VPAD // _NSUB

    @functools.partial(
        pl.kernel,
        mesh=mesh,
        out_type=jax.ShapeDtypeStruct((_VPAD,), jnp.float32),
        compiler_params=pltpu.CompilerParams(
            needs_layout_passes=False, use_tc_tiling_on_sc=False),
        scratch_types=[
            pltpu.VMEM((idx_per_sub,), jnp.int32),
            pltpu.VMEM((16 * _VPAD,), jnp.float32),
            pltpu.VMEM((_VPAD,), jnp.float32),
            pltpu.VMEM((_NSUB, _VPAD), jnp.float32),
            pltpu.VMEM((stripe,), jnp.float32),
            pltpu.VMEM_SHARED((_NSUB, _VPAD), jnp.float32),
            pltpu.SemaphoreType.DMA,
        ],
    )
    def bincount(idx_hbm, wc_hbm, out_hbm, idx_v, hist2, hist1, parts, ostripe,
                 shared, sem):
        core = lax.axis_index("c")
        sub = lax.axis_index("s")

        @pl.when(core == 0)
        def _():
            cp = pltpu.make_async_copy(
                idx_hbm.at[pl.ds(sub * idx_per_sub, idx_per_sub)], idx_v, sem)
            cp.start()
            zeros16 = jnp.zeros((16,), jnp.float32)

            def zero_body(g, c):
                for u in range(8):
                    hist2[pl.ds((g * 8 + u) * 16, 16)] = zeros16
                return c
            lax.fori_loop(0, _VPAD // 8, zero_body, 0)
            cp.wait()

            lane = lax.iota(jnp.int32, 16) * _VPAD
            ones16 = jnp.full((16,), 1.0, jnp.float32)

            def scat_body(g, c):
                for u in range(4):
                    iv = idx_v[pl.ds((g * 4 + u) * 16, 16)]
                    plsc.addupdate_scatter(hist2, [lane + iv], ones16)
                return c
            lax.fori_loop(0, idx_per_sub // 64, scat_body, 0)

            def red_body(c, carry):
                acc = hist2[pl.ds(c * 16, 16)]
                for l in range(1, 16):
                    acc = acc + hist2[pl.ds(l * _VPAD + c * 16, 16)]
                hist1[pl.ds(c * 16, 16)] = acc
                return carry
            lax.fori_loop(0, _VPAD // 16, red_body, 0)

            pltpu.sync_copy(hist1, shared.at[sub])
            plsc.subcore_barrier()
            pltpu.sync_copy(shared, parts)
            pltpu.sync_copy(wc_hbm.at[pl.ds(sub * stripe, stripe)], ostripe)

            def comb_body(r, carry):
                acc = ostripe[pl.ds(r * 16, 16)]
                for t in range(_NSUB):
                    acc = acc + parts[t, pl.ds(sub * stripe + r * 16, 16)]
                ostripe[pl.ds(r * 16, 16)] = acc
                return carry
            lax.fori_loop(0, stripe // 16, comb_body, 0)
            pltpu.sync_copy(ostripe, out_hbm.at[pl.ds(sub * stripe, stripe)])

    return bincount(indices, wc1d)


def kernel(utterances, word_counts):
    batch, agents, vocab = utterances.shape
    nrows = batch * agents
    utt_t = jnp.transpose(utterances, (1, 2, 0))
    indices = _tc_argmax(utt_t)
    wc1d = jnp.pad(word_counts, (0, _VPAD - vocab))
    hist = _sc_bincount(indices, wc1d, nrows // _NSUB)
    return hist[:vocab]


# speedup vs baseline: 1.0740x; 1.0115x over previous
import functools

import jax
import jax.numpy as jnp
from jax import lax
from jax.experimental import pallas as pl
from jax.experimental.pallas import tpu as pltpu
from jax.experimental.pallas import tpu_sc as plsc

_VOCAB = 1000
_VPAD = 1024
_ROWS_PER_BLOCK = 2048
_NSUB = 16


_LANES_PER_BLOCK = 2048


def _argmax_body(x0_ref, x1_ref, out_ref):
    _, vocab, bl = x0_ref.shape
    ids = lax.broadcasted_iota(jnp.int32, (vocab, bl), 0)
    for a, ref in enumerate((x0_ref, x1_ref)):
        xa = ref[0]
        m = jnp.max(xa, axis=0, keepdims=True)
        idx = jnp.min(jnp.where(xa == m, ids, vocab), axis=0)
        out_ref[pl.ds(a * bl, bl)] = idx


def _tc_argmax(utt_t):
    agents, vocab, batch = utt_t.shape
    nblk = batch // _LANES_PER_BLOCK
    blk = (1, vocab, _LANES_PER_BLOCK)
    out = pl.pallas_call(
        _argmax_body,
        grid=(nblk,),
        in_specs=[pl.BlockSpec(blk, lambda i: (0, 0, i)),
                  pl.BlockSpec(blk, lambda i: (1, 0, i))],
        out_specs=pl.BlockSpec((agents * _LANES_PER_BLOCK,), lambda i: (i,)),
        out_shape=jax.ShapeDtypeStruct((agents * batch,), jnp.int32),
        compiler_params=pltpu.CompilerParams(
            vmem_limit_bytes=100 * 1024 * 1024),
    )(utt_t, utt_t)
    return out


def _sc_bincount(indices, word_counts, idx_per_sub):
    mesh = plsc.VectorSubcoreMesh(core_axis_name="c", subcore_axis_name="s")
    stripe = _VPAD // _NSUB
    last = _NSUB - 1
    tail = _VOCAB - last * stripe

    @functools.partial(
        pl.kernel,
        mesh=mesh,
        out_type=jax.ShapeDtypeStruct((_VOCAB,), jnp.float32),
        compiler_params=pltpu.CompilerParams(
            needs_layout_passes=False, use_tc_tiling_on_sc=False),
        scratch_types=[
            pltpu.VMEM((idx_per_sub,), jnp.int32),
            pltpu.VMEM((16 * _VPAD,), jnp.float32),
            pltpu.VMEM((_VPAD,), jnp.float32),
            pltpu.VMEM((_NSUB, _VPAD), jnp.float32),
            pltpu.VMEM((stripe,), jnp.float32),
            pltpu.VMEM_SHARED((_NSUB, _VPAD), jnp.float32),
            pltpu.SemaphoreType.DMA,
        ],
    )
    def bincount(idx_hbm, wc_hbm, out_hbm, idx_v, hist2, hist1, parts, ostripe,
                 shared, sem):
        core = lax.axis_index("c")
        sub = lax.axis_index("s")

        @pl.when(core == 0)
        def _():
            cp = pltpu.make_async_copy(
                idx_hbm.at[pl.ds(sub * idx_per_sub, idx_per_sub)], idx_v, sem)
            cp.start()
            zeros16 = jnp.zeros((16,), jnp.float32)

            def zero_body(g, c):
                for u in range(8):
                    hist2[pl.ds((g * 8 + u) * 16, 16)] = zeros16
                return c
            lax.fori_loop(0, _VPAD // 8, zero_body, 0)
            cp.wait()

            lane = lax.iota(jnp.int32, 16) * _VPAD
            ones16 = jnp.full((16,), 1.0, jnp.float32)

            def scat_body(g, c):
                for u in range(4):
                    iv = idx_v[pl.ds((g * 4 + u) * 16, 16)]
                    plsc.addupdate_scatter(hist2, [lane + iv], ones16)
                return c
            lax.fori_loop(0, idx_per_sub // 64, scat_body, 0)

            def red_body(c, carry):
                acc = hist2[pl.ds(c * 16, 16)]
                for l in range(1, 16):
                    acc = acc + hist2[pl.ds(l * _VPAD + c * 16, 16)]
                hist1[pl.ds(c * 16, 16)] = acc
                return carry
            lax.fori_loop(0, _VPAD // 16, red_body, 0)

            pltpu.sync_copy(hist1, shared.at[sub])
            plsc.subcore_barrier()
            pltpu.sync_copy(shared, parts)

            @pl.when(sub != last)
            def _():
                pltpu.sync_copy(wc_hbm.at[pl.ds(sub * stripe, stripe)],
                                ostripe)

            @pl.when(sub == last)
            def _():
                ostripe[pl.ds(tail, 16)] = zeros16
                ostripe[pl.ds(stripe - 16, 16)] = zeros16
                pltpu.sync_copy(wc_hbm.at[pl.ds(last * stripe, tail)],
                                ostripe.at[pl.ds(0, tail)])

            def comb_body(r, carry):
                acc = ostripe[pl.ds(r * 16, 16)]
                for t in range(_NSUB):
                    acc = acc + parts[t, pl.ds(sub * stripe + r * 16, 16)]
                ostripe[pl.ds(r * 16, 16)] = acc
                return carry
            lax.fori_loop(0, stripe // 16, comb_body, 0)

            @pl.when(sub != last)
            def _():
                pltpu.sync_copy(ostripe,
                                out_hbm.at[pl.ds(sub * stripe, stripe)])

            @pl.when(sub == last)
            def _():
                pltpu.sync_copy(ostripe.at[pl.ds(0, tail)],
                                out_hbm.at[pl.ds(last * stripe, tail)])

    return bincount(indices, word_counts)


def kernel(utterances, word_counts):
    batch, agents, vocab = utterances.shape
    nrows = batch * agents
    utt_t = jnp.transpose(utterances, (1, 2, 0))
    indices = _tc_argmax(utt_t)
    return _sc_bincount(indices, word_counts, nrows // _NSUB)
